# baseline (speedup 1.0000x reference)
import functools

import jax
import jax.numpy as jnp
from jax import lax
from jax.experimental import pallas as pl
from jax.experimental.pallas import tpu as pltpu
from jax.experimental.pallas import tpu_sc as plsc

_NUSER = 5000
_NITEM = 5000
_N = 10000
_D = 128
_E = 160000
_NC, _NS, _LANES = 2, 16, 16
_NW = _NC * _NS
_EW = _E // _NW
_CH = 128
_NCH = 40
_EPAD = _NCH * _CH
_IOFF = 5120
_PK = 10240
_YPAD = _N + 112
_ROWS_PER_SUB = _YPAD // _NS
_COLS_PER_SUB = _PK // _NS
_MMW = 160
_MMCLAMP = _NITEM - _MMW

_mesh = plsc.VectorSubcoreMesh(core_axis_name="c", subcore_axis_name="s")

_GDN = lax.GatherDimensionNumbers(
    offset_dims=(), collapsed_slice_dims=(0,), start_index_map=(0,))


def _lane_perm(a, idx):
    return lax.gather(a, idx[:, None], _GDN, (1,),
                      mode=lax.GatherScatterMode.PROMISE_IN_BOUNDS)


def _mm_ss_body(x_ref, w_ref, xw_ref, ss_ref):
    x = x_ref[...]
    xw_ref[...] = jnp.dot(x, w_ref[...], preferred_element_type=jnp.float32)
    ss_ref[...] = jnp.maximum(jnp.sum(x * x, axis=1, keepdims=True), 1e-24)


def _mm_ss(x, w):
    rows = x.shape[0]
    blk = 1000
    return pl.pallas_call(
        _mm_ss_body,
        grid=(rows // blk,),
        in_specs=[
            pl.BlockSpec((blk, _D), lambda i: (i, 0)),
            pl.BlockSpec((_D, _D), lambda i: (0, 0)),
        ],
        out_specs=[
            pl.BlockSpec((blk, _D), lambda i: (i, 0)),
            pl.BlockSpec((blk, 1), lambda i: (i, 0)),
        ],
        out_shape=[
            jax.ShapeDtypeStruct((rows, _D), jnp.float32),
            jax.ShapeDtypeStruct((rows, 1), jnp.float32),
        ],
    )(x, w)


def _yprep_body(deg_ref, xw_ref, b_ref, y_ref, self_ref, dis_ref):
    dis = lax.rsqrt(deg_ref[...] + 1.0)
    xw = xw_ref[...]
    y_ref[...] = dis * xw
    self_ref[...] = dis * dis * xw + b_ref[...]
    dis_ref[...] = dis


def _yprep(deg2d, xw, b):
    blk = 1000
    return pl.pallas_call(
        _yprep_body,
        grid=(_N // blk,),
        in_specs=[
            pl.BlockSpec((blk, 1), lambda i: (i, 0)),
            pl.BlockSpec((blk, _D), lambda i: (i, 0)),
            pl.BlockSpec((1, _D), lambda i: (0, 0)),
        ],
        out_specs=[
            pl.BlockSpec((blk, _D), lambda i: (i, 0)),
            pl.BlockSpec((blk, _D), lambda i: (i, 0)),
            pl.BlockSpec((blk, 1), lambda i: (i, 0)),
        ],
        out_shape=[
            jax.ShapeDtypeStruct((_N, _D), jnp.float32),
            jax.ShapeDtypeStruct((_N, _D), jnp.float32),
            jax.ShapeDtypeStruct((_N, 1), jnp.float32),
        ],
    )(deg2d, xw, b.reshape(1, _D))


def _comb_body(dis_ref, a0_ref, a1_ref, self_ref, out_ref):
    out_ref[...] = dis_ref[...] * (a0_ref[...] + a1_ref[...]) + self_ref[...]


def _combine(dis, a0, a1, selfo):
    blk = 1000
    return pl.pallas_call(
        _comb_body,
        grid=(_N // blk,),
        in_specs=[
            pl.BlockSpec((blk, 1), lambda i: (i, 0)),
            pl.BlockSpec((blk, _D), lambda i: (i, 0)),
            pl.BlockSpec((blk, _D), lambda i: (i, 0)),
            pl.BlockSpec((blk, _D), lambda i: (i, 0)),
        ],
        out_specs=pl.BlockSpec((blk, _D), lambda i: (i, 0)),
        out_shape=jax.ShapeDtypeStruct((_N, _D), jnp.float32),
    )(dis, a0, a1, selfo)


def _final_body(alls_ref, h_ref, out_ref):
    i = pl.program_id(0)
    x = alls_ref[...] * (1.0 / 3.0)
    h = h_ref[...]
    nrm = jnp.maximum(jnp.sqrt(jnp.sum(h * h, axis=1, keepdims=True)), 1e-12)
    item_part = jnp.where(i == 1, 1.0, 0.0)
    out_ref[...] = x + item_part * (h / nrm)


def _final(alls, h):
    return pl.pallas_call(
        _final_body,
        grid=(2,),
        in_specs=[
            pl.BlockSpec((_NUSER, _D), lambda i: (i, 0)),
            pl.BlockSpec((_NITEM, _D), lambda i: (0, 0)),
        ],
        out_specs=pl.BlockSpec((_NUSER, _D), lambda i: (i, 0)),
        out_shape=jax.ShapeDtypeStruct((_N, _D), jnp.float32),
    )(alls, h)


def _mm_body(h_hbm, cols_hbm, vals_hbm, out_hbm,
             cols_v, vals_v, bufA, bufB, hout_v, semA, semB):
    c = lax.axis_index("c")
    s = lax.axis_index("s")
    wid = s * _NC + c
    i0 = pl.multiple_of(jnp.minimum(wid * _MMW, _MMCLAMP), 8)
    pltpu.sync_copy(cols_hbm.at[pl.ds(i0, _MMW)], cols_v)
    pltpu.sync_copy(vals_hbm.at[pl.ds(i0, _MMW)], vals_v)

    def compute(li, buf):
        v0 = vals_v[li, pl.ds(0, 16)]
        v1 = vals_v[li, pl.ds(16, 16)]
        for g in range(8):
            sl = pl.ds(g * 16, 16)
            acc = v0[0] * buf[0, sl]
            for k in range(1, 16):
                acc = acc + v0[k] * buf[k, sl]
            for k in range(4):
                acc = acc + v1[k] * buf[16 + k, sl]
            hout_v[li, sl] = acc

    def gather(li, buf, sem):
        li = jnp.minimum(li, _MMW - 1)
        return pltpu.async_copy(h_hbm.at[cols_v.at[li, pl.ds(0, 20)]],
                                buf, sem)

    gather(0, bufA, semA)

    def item2(t, _):
        la = 2 * t
        gb = gather(la + 1, bufB, semB)
        pltpu.make_async_copy(h_hbm.at[cols_v.at[0, pl.ds(0, 20)]],
                              bufA, semA).wait()
        compute(la, bufA)
        ga = gather(la + 2, bufA, semA)
        del ga
        gb.wait()
        compute(la + 1, bufB)
        return 0

    lax.fori_loop(0, _MMW // 2, item2, 0)
    pltpu.make_async_copy(h_hbm.at[cols_v.at[0, pl.ds(0, 20)]],
                          bufA, semA).wait()
    pltpu.sync_copy(hout_v, out_hbm.at[pl.ds(i0, _MMW)])


@functools.partial(
    pl.kernel,
    out_type=jax.ShapeDtypeStruct((_NITEM, _D), jnp.float32),
    mesh=_mesh,
    compiler_params=pltpu.CompilerParams(needs_layout_passes=False),
    scratch_types=[
        pltpu.VMEM((_MMW, 32), jnp.int32),
        pltpu.VMEM((_MMW, 32), jnp.float32),
        pltpu.VMEM((20, _D), jnp.float32),
        pltpu.VMEM((20, _D), jnp.float32),
        pltpu.VMEM((_MMW, _D), jnp.float32),
        pltpu.SemaphoreType.DMA,
        pltpu.SemaphoreType.DMA,
    ],
)
def _mm_prop(h_hbm, cols_hbm, vals_hbm, out_hbm, *rest):
    _mm_body(h_hbm, cols_hbm, vals_hbm, out_hbm, *rest)


_MCH = 32
_MNCH = _EPAD // _MCH


def _mask_deg_body(ego_hbm, ss_hbm, eu_hbm, ei_hbm, mask_hbm, deg_hbm,
                   eu_v, eip_v, mask_v, ss_v, degp_v, bufU0, bufI0,
                   bufU1, bufI1, s_v, red_v, degout_v, shared,
                   semU0, semI0, semU1, semI1):
    c = lax.axis_index("c")
    s = lax.axis_index("s")
    wid = s * _NC + c
    base = pl.multiple_of(wid * _EW, 8)
    zi = jnp.zeros((16,), jnp.int32)
    zf = jnp.zeros((16,), jnp.float32)
    for t in range(8):
        off = _EPAD - 128 + t * 16
        eu_v[pl.ds(off, 16)] = zi
        eip_v[pl.ds(off, 16)] = zi
    pltpu.sync_copy(eu_hbm.at[pl.ds(base, _EW)], eu_v.at[pl.ds(0, _EW)])
    pltpu.sync_copy(ei_hbm.at[pl.ds(base, _EW)], eip_v.at[pl.ds(0, _EW)])
    pltpu.sync_copy(ss_hbm, ss_v)

    def zdeg(k, _):
        degp_v[pl.ds(k * 16, 16)] = zf
        return 0

    lax.fori_loop(0, _PK // 16, zdeg, 0)

    def addoff(k, _):
        sl = pl.ds(k * 16, 16)
        eip_v[sl] = eip_v[sl] + _NUSER
        return 0

    lax.fori_loop(0, _EPAD // 16, addoff, 0)
    lanes = lax.iota(jnp.int32, 16)
    l16 = lanes * 16

    def gather(j, bufU, bufI, semU, semI):
        j = jnp.minimum(j, _MNCH - 1)
        e0 = pl.multiple_of(j * _MCH, _MCH)
        pltpu.async_copy(ego_hbm.at[eu_v.at[pl.ds(e0, _MCH)]], bufU, semU)
        pltpu.async_copy(ego_hbm.at[eip_v.at[pl.ds(e0, _MCH)]], bufI, semI)

    def drain(bufU, bufI, semU, semI):
        pltpu.make_async_copy(ego_hbm.at[eu_v.at[pl.ds(0, _MCH)]],
                              bufU, semU).wait()
        pltpu.make_async_copy(ego_hbm.at[eip_v.at[pl.ds(0, _MCH)]],
                              bufI, semI).wait()

    def process(j, bufU, bufI):
        e0 = pl.multiple_of(j * _MCH, _MCH)
        for k in range(_MCH // 16):
            for l in range(16):
                el = k * 16 + l
                p = [bufU[el, pl.ds(g * 16, 16)] * bufI[el, pl.ds(g * 16, 16)]
                     for g in range(8)]
                while len(p) > 1:
                    p = [p[i] + p[i + 1] for i in range(0, len(p), 2)]
                s_v[pl.ds(l * 16, 16)] = p[0]
            q = [plsc.load_gather(s_v, [l16 + jj]) for jj in range(16)]
            while len(q) > 1:
                q = [q[i] + q[i + 1] for i in range(0, len(q), 2)]
            dvec = q[0]
            o = e0 + k * 16
            eu = eu_v[pl.ds(o, 16)]
            eip = eip_v[pl.ds(o, 16)]
            ssu = plsc.load_gather(ss_v, [eu])
            ssi = plsc.load_gather(ss_v, [eip + (_IOFF - _NUSER)])
            m = (dvec >= 0.0) & (dvec * dvec >= 0.0025 * ssu * ssi)
            m = m & (o + lanes < _EW)
            mf = jnp.where(m, 1.0, 0.0)
            mask_v[pl.ds(o, 16)] = mf
            plsc.addupdate_scatter(degp_v, [eu], mf)
            plsc.addupdate_scatter(degp_v, [eip + (_IOFF - _NUSER)], mf)

    gather(0, bufU0, bufI0, semU0, semI0)

    def pair(t, _):
        j0 = 2 * t
        gather(j0 + 1, bufU1, bufI1, semU1, semI1)
        drain(bufU0, bufI0, semU0, semI0)
        process(j0, bufU0, bufI0)
        gather(j0 + 2, bufU0, bufI0, semU0, semI0)
        drain(bufU1, bufI1, semU1, semI1)
        process(j0 + 1, bufU1, bufI1)
        return 0

    lax.fori_loop(0, _MNCH // 2, pair, 0)
    drain(bufU0, bufI0, semU0, semI0)
    pltpu.sync_copy(mask_v.at[pl.ds(0, _EW)], mask_hbm.at[pl.ds(base, _EW)])
    pltpu.sync_copy(degp_v, shared.at[s])
    plsc.subcore_barrier()
    col0 = pl.multiple_of(s * _COLS_PER_SUB, 128)
    pltpu.sync_copy(shared.at[pl.ds(0, _NS), pl.ds(col0, _COLS_PER_SUB)], red_v)

    def red(k, _):
        sl = pl.ds(k * 16, 16)
        a = red_v[0, sl]
        for t2 in range(1, _NS):
            a = a + red_v[t2, sl]
        degout_v[sl] = a
        return 0

    lax.fori_loop(0, _COLS_PER_SUB // 16, red, 0)
    pltpu.sync_copy(
        degout_v,
        deg_hbm.at[pl.ds(pl.multiple_of(c * _PK + col0, 128), _COLS_PER_SUB)])


@functools.partial(
    pl.kernel,
    out_type=[
        jax.ShapeDtypeStruct((_E + _CH,), jnp.float32),
        jax.ShapeDtypeStruct((_NC * _PK,), jnp.float32),
    ],
    mesh=_mesh,
    compiler_params=pltpu.CompilerParams(needs_layout_passes=False),
    scratch_types=[
        pltpu.VMEM((_EPAD,), jnp.int32),
        pltpu.VMEM((_EPAD,), jnp.int32),
        pltpu.VMEM((_EPAD,), jnp.float32),
        pltpu.VMEM((_PK,), jnp.float32),
        pltpu.VMEM((_PK,), jnp.float32),
        pltpu.VMEM((_MCH, _D), jnp.float32),
        pltpu.VMEM((_MCH, _D), jnp.float32),
        pltpu.VMEM((_MCH, _D), jnp.float32),
        pltpu.VMEM((_MCH, _D), jnp.float32),
        pltpu.VMEM((256,), jnp.float32),
        pltpu.VMEM((_NS, _COLS_PER_SUB), jnp.float32),
        pltpu.VMEM((_COLS_PER_SUB,), jnp.float32),
        pltpu.VMEM_SHARED((_NS, _PK), jnp.float32),
        pltpu.SemaphoreType.DMA,
        pltpu.SemaphoreType.DMA,
        pltpu.SemaphoreType.DMA,
        pltpu.SemaphoreType.DMA,
    ],
)
def _mask_deg(ego_hbm, ss_hbm, eu_hbm, ei_hbm, mask_hbm, deg_hbm, *rest):
    _mask_deg_body(ego_hbm, ss_hbm, eu_hbm, ei_hbm, mask_hbm, deg_hbm, *rest)


def _gcn_scatter_body(y_hbm, eu_hbm, ei_hbm, mask_hbm, zeros_hbm, out_hbm,
                      eubuf, eibuf, mbuf, srcu_v, srci_v, d1_v, d2_v,
                      bufA, bufB, acc_sh, semA, semB, semSA, semSB):
    c = lax.axis_index("c")
    s = lax.axis_index("s")
    wid = s * _NC + c
    base = pl.multiple_of(wid * _EW, 8)
    rows0 = pl.multiple_of(s * _ROWS_PER_SUB, 8)
    pltpu.sync_copy(zeros_hbm.at[pl.ds(rows0, _ROWS_PER_SUB)],
                    acc_sh.at[pl.ds(rows0, _ROWS_PER_SUB)])
    lanes = lax.iota(jnp.int32, 16)

    def build(j, slot):
        j = jnp.minimum(j, _NCH - 1)
        e0 = pl.multiple_of(j * _CH, _CH)
        pltpu.sync_copy(mask_hbm.at[pl.ds(base + e0, _CH)], mbuf)
        pltpu.sync_copy(eu_hbm.at[pl.ds(base + e0, _CH)], eubuf)
        pltpu.sync_copy(ei_hbm.at[pl.ds(base + e0, _CH)], eibuf)
        for k in range(_CH // 16):
            sl = pl.ds(k * 16, 16)
            eu = eubuf[sl]
            ei = eibuf[sl] + _NUSER
            m = (mbuf[sl] > 0.5) & (e0 + k * 16 + lanes < _EW)
            pad = _N + ((wid + k) * 16) % 112 + lanes
            srcu_v[slot, sl] = jnp.where(m, eu, pad)
            srci_v[slot, sl] = jnp.where(m, ei, pad)
            d1_v[slot, sl] = ei
            d2_v[slot, sl] = eu

    def gathers(slot):
        pltpu.async_copy(y_hbm.at[srcu_v.at[slot]], bufA, semA)
        pltpu.async_copy(y_hbm.at[srci_v.at[slot]], bufB, semB)

    def gwait():
        pltpu.make_async_copy(y_hbm.at[srcu_v.at[0]], bufA, semA).wait()
        pltpu.make_async_copy(y_hbm.at[srci_v.at[0]], bufB, semB).wait()

    def scatters(slot):
        pltpu.async_copy(bufA, acc_sh.at[d1_v.at[slot]], semSA, add=True)
        pltpu.async_copy(bufB, acc_sh.at[d2_v.at[slot]], semSB, add=True)

    def swait():
        pltpu.make_async_copy(bufA, acc_sh.at[d1_v.at[0]], semSA).wait()
        pltpu.make_async_copy(bufB, acc_sh.at[d2_v.at[0]], semSB).wait()

    plsc.subcore_barrier()
    build(0, 0)
    gathers(0)

    def step2(t, _):
        j0 = 2 * t
        build(j0 + 1, 1)
        gwait()
        scatters(0)
        swait()
        gathers(1)
        build(j0 + 2, 0)
        gwait()
        scatters(1)
        swait()
        gathers(0)
        return 0

    lax.fori_loop(0, _NCH // 2, step2, 0)
    gwait()
    plsc.subcore_barrier()
    pltpu.sync_copy(acc_sh.at[pl.ds(rows0, _ROWS_PER_SUB)],
                    out_hbm.at[c, pl.ds(rows0, _ROWS_PER_SUB)])


@functools.partial(
    pl.kernel,
    out_type=jax.ShapeDtypeStruct((_NC, _YPAD, _D), jnp.float32),
    mesh=_mesh,
    compiler_params=pltpu.CompilerParams(needs_layout_passes=False),
    scratch_types=[
        pltpu.VMEM((_CH,), jnp.int32),
        pltpu.VMEM((_CH,), jnp.int32),
        pltpu.VMEM((_CH,), jnp.float32),
        pltpu.VMEM((2, _CH), jnp.int32),
        pltpu.VMEM((2, _CH), jnp.int32),
        pltpu.VMEM((2, _CH), jnp.int32),
        pltpu.VMEM((2, _CH), jnp.int32),
        pltpu.VMEM((_CH, _D), jnp.float32),
        pltpu.VMEM((_CH, _D), jnp.float32),
        pltpu.VMEM_SHARED((_YPAD, _D), jnp.float32),
        pltpu.SemaphoreType.DMA,
        pltpu.SemaphoreType.DMA,
        pltpu.SemaphoreType.DMA,
        pltpu.SemaphoreType.DMA,
    ],
)
def _gcn_scatter(y_hbm, eu_hbm, ei_hbm, mask_hbm, zeros_hbm, out_hbm, *rest):
    _gcn_scatter_body(y_hbm, eu_hbm, ei_hbm, mask_hbm, zeros_hbm, out_hbm,
                      *rest)


def _gcn_layer(ego, W, b, e_user, e_item, eu_pad, ei_pad, zeros_pad):
    xw, ss = _mm_ss(ego, W)
    ss1 = ss[:, 0]
    ss_packed = jnp.concatenate([
        ss1[:_NUSER], jnp.ones((_IOFF - _NUSER,), jnp.float32),
        ss1[_NUSER:], jnp.ones((_PK - _IOFF - _NITEM,), jnp.float32),
    ])
    mask, degp_flat = _mask_deg(ego, ss_packed, e_user, e_item)
    degp = degp_flat.reshape(_NC, _PK)
    degsum = degp[0] + degp[1]
    deg2d = jnp.concatenate(
        [degsum[:_NUSER], degsum[_IOFF:_IOFF + _NITEM]]).reshape(_N, 1)
    y, selfo, dis = _yprep(deg2d, xw, b)
    y_full = jnp.concatenate([y, jnp.zeros((_YPAD - _N, _D), jnp.float32)], 0)
    acc = _gcn_scatter(y_full, eu_pad, ei_pad, mask, zeros_pad)
    return _combine(dis, acc[0, :_N], acc[1, :_N], selfo)


def kernel(user_emb, item_emb, img_W, img_b, txt_W, txt_b, Wg0, bg0, Wg1, bg1,
           Wm0, bm0, Wm1, bm1, v_feat, t_feat, mm_vals, e_user, e_item,
           mm_rows, mm_cols):
    colsA = mm_cols[:_NITEM * 10].reshape(_NITEM, 10)
    colsB = mm_cols[_NITEM * 10:].reshape(_NITEM, 10)
    valsA = mm_vals[:_NITEM * 10].reshape(_NITEM, 10)
    valsB = mm_vals[_NITEM * 10:].reshape(_NITEM, 10)
    spread = (jnp.arange(_NITEM, dtype=jnp.int32)[:, None]
              + jnp.arange(12, dtype=jnp.int32)[None, :]) % _NITEM
    cols32 = jnp.concatenate([colsA, colsB, spread], axis=1)
    vals32 = jnp.concatenate(
        [valsA, valsB, jnp.zeros((_NITEM, 12), jnp.float32)], axis=1)
    h = item_emb
    h = _mm_prop(h, cols32, vals32)
    h = _mm_prop(h, cols32, vals32)

    zeros_pad = jnp.zeros((_YPAD, _D), jnp.float32)
    eu_pad = jnp.concatenate([e_user, jnp.zeros((256,), jnp.int32)])
    ei_pad = jnp.concatenate([e_item, jnp.zeros((256,), jnp.int32)])
    ego = jnp.concatenate([user_emb, item_emb], 0)
    alls = ego
    for (W, b) in ((Wg0, bg0), (Wg1, bg1)):
        ego = _gcn_layer(ego, W, b, e_user, e_item, eu_pad, ei_pad,
                         zeros_pad)
        alls = alls + ego
    return _final(alls, h)


# speedup vs baseline: 1.0466x; 1.0466x over previous
import functools

import jax
import jax.numpy as jnp
from jax import lax
from jax.experimental import pallas as pl
from jax.experimental.pallas import tpu as pltpu
from jax.experimental.pallas import tpu_sc as plsc

_NUSER = 5000
_NITEM = 5000
_N = 10000
_D = 128
_E = 160000
_NC, _NS, _LANES = 2, 16, 16
_NW = _NC * _NS
_EW = _E // _NW
_CH = 128
_NCH = 40
_EPAD = _NCH * _CH
_IOFF = 5120
_PK = 10240
_YPAD = _N + 112
_ROWS_PER_SUB = _YPAD // _NS
_COLS_PER_SUB = _PK // _NS
_MMW = 160
_MMCLAMP = _NITEM - _MMW

_mesh = plsc.VectorSubcoreMesh(core_axis_name="c", subcore_axis_name="s")

_GDN = lax.GatherDimensionNumbers(
    offset_dims=(), collapsed_slice_dims=(0,), start_index_map=(0,))


def _lane_perm(a, idx):
    return lax.gather(a, idx[:, None], _GDN, (1,),
                      mode=lax.GatherScatterMode.PROMISE_IN_BOUNDS)


def _mm_ss_body(x_ref, w_ref, xw_ref, ss_ref):
    x = x_ref[...]
    xw_ref[...] = jnp.dot(x, w_ref[...], preferred_element_type=jnp.float32)
    ss_ref[...] = jnp.maximum(jnp.sum(x * x, axis=1, keepdims=True), 1e-24)


def _mm_ss(x, w):
    rows = x.shape[0]
    blk = 1000
    return pl.pallas_call(
        _mm_ss_body,
        grid=(rows // blk,),
        in_specs=[
            pl.BlockSpec((blk, _D), lambda i: (i, 0)),
            pl.BlockSpec((_D, _D), lambda i: (0, 0)),
        ],
        out_specs=[
            pl.BlockSpec((blk, _D), lambda i: (i, 0)),
            pl.BlockSpec((blk, 1), lambda i: (i, 0)),
        ],
        out_shape=[
            jax.ShapeDtypeStruct((rows, _D), jnp.float32),
            jax.ShapeDtypeStruct((rows, 1), jnp.float32),
        ],
    )(x, w)


def _yprep_body(deg_ref, xw_ref, b_ref, y_ref, self_ref, dis_ref):
    dis = lax.rsqrt(deg_ref[...] + 1.0)
    xw = xw_ref[...]
    y_ref[...] = dis * xw
    self_ref[...] = dis * dis * xw + b_ref[...]
    dis_ref[...] = dis


def _yprep(deg2d, xw, b):
    blk = 1000
    return pl.pallas_call(
        _yprep_body,
        grid=(_N // blk,),
        in_specs=[
            pl.BlockSpec((blk, 1), lambda i: (i, 0)),
            pl.BlockSpec((blk, _D), lambda i: (i, 0)),
            pl.BlockSpec((1, _D), lambda i: (0, 0)),
        ],
        out_specs=[
            pl.BlockSpec((blk, _D), lambda i: (i, 0)),
            pl.BlockSpec((blk, _D), lambda i: (i, 0)),
            pl.BlockSpec((blk, 1), lambda i: (i, 0)),
        ],
        out_shape=[
            jax.ShapeDtypeStruct((_N, _D), jnp.float32),
            jax.ShapeDtypeStruct((_N, _D), jnp.float32),
            jax.ShapeDtypeStruct((_N, 1), jnp.float32),
        ],
    )(deg2d, xw, b.reshape(1, _D))


def _comb_body(dis_ref, a0_ref, a1_ref, self_ref, out_ref):
    out_ref[...] = dis_ref[...] * (a0_ref[...] + a1_ref[...]) + self_ref[...]


def _combine(dis, a0, a1, selfo):
    blk = 1000
    return pl.pallas_call(
        _comb_body,
        grid=(_N // blk,),
        in_specs=[
            pl.BlockSpec((blk, 1), lambda i: (i, 0)),
            pl.BlockSpec((blk, _D), lambda i: (i, 0)),
            pl.BlockSpec((blk, _D), lambda i: (i, 0)),
            pl.BlockSpec((blk, _D), lambda i: (i, 0)),
        ],
        out_specs=pl.BlockSpec((blk, _D), lambda i: (i, 0)),
        out_shape=jax.ShapeDtypeStruct((_N, _D), jnp.float32),
    )(dis, a0, a1, selfo)


def _final_body(alls_ref, h_ref, out_ref):
    i = pl.program_id(0)
    x = alls_ref[...] * (1.0 / 3.0)
    h = h_ref[...]
    nrm = jnp.maximum(jnp.sqrt(jnp.sum(h * h, axis=1, keepdims=True)), 1e-12)
    item_part = jnp.where(i == 1, 1.0, 0.0)
    out_ref[...] = x + item_part * (h / nrm)


def _final(alls, h):
    return pl.pallas_call(
        _final_body,
        grid=(2,),
        in_specs=[
            pl.BlockSpec((_NUSER, _D), lambda i: (i, 0)),
            pl.BlockSpec((_NITEM, _D), lambda i: (0, 0)),
        ],
        out_specs=pl.BlockSpec((_NUSER, _D), lambda i: (i, 0)),
        out_shape=jax.ShapeDtypeStruct((_N, _D), jnp.float32),
    )(alls, h)


def _mm_body(h_hbm, cols_hbm, vals_hbm, out_hbm,
             cols_v, vals_v, b0, b1, b2, b3, hout_v, s0, s1, s2, s3):
    c = lax.axis_index("c")
    s = lax.axis_index("s")
    wid = s * _NC + c
    i0 = pl.multiple_of(jnp.minimum(wid * _MMW, _MMCLAMP), 8)
    pltpu.sync_copy(cols_hbm.at[pl.ds(i0, _MMW)], cols_v)
    pltpu.sync_copy(vals_hbm.at[pl.ds(i0, _MMW)], vals_v)

    def compute(li, buf):
        v0 = vals_v[li, pl.ds(0, 16)]
        v1 = vals_v[li, pl.ds(16, 16)]
        for g in range(8):
            sl = pl.ds(g * 16, 16)
            p = [v0[k] * buf[k, sl] for k in range(16)]
            p += [v1[k] * buf[16 + k, sl] for k in range(4)]
            while len(p) > 1:
                p = [p[i] + p[i + 1] for i in range(0, len(p), 2)]                     if len(p) % 2 == 0 else                     [p[i] + p[i + 1] for i in range(0, len(p) - 1, 2)] + [p[-1]]
            hout_v[li, sl] = p[0]

    def gather(li, buf, sem):
        li = jnp.minimum(li, _MMW - 1)
        return pltpu.async_copy(h_hbm.at[cols_v.at[li, pl.ds(0, 20)]],
                                buf, sem)

    def drain(buf, sem):
        pltpu.make_async_copy(h_hbm.at[cols_v.at[0, pl.ds(0, 20)]],
                              buf, sem).wait()

    gather(0, b0, s0)
    gather(1, b1, s1)
    gather(2, b2, s2)

    def item4(t, _):
        li = 4 * t
        gather(li + 3, b3, s3)
        drain(b0, s0)
        compute(li, b0)
        gather(li + 4, b0, s0)
        drain(b1, s1)
        compute(li + 1, b1)
        gather(li + 5, b1, s1)
        drain(b2, s2)
        compute(li + 2, b2)
        gather(li + 6, b2, s2)
        drain(b3, s3)
        compute(li + 3, b3)
        return 0

    lax.fori_loop(0, _MMW // 4, item4, 0)
    drain(b0, s0)
    drain(b1, s1)
    drain(b2, s2)
    pltpu.sync_copy(hout_v, out_hbm.at[pl.ds(i0, _MMW)])


@functools.partial(
    pl.kernel,
    out_type=jax.ShapeDtypeStruct((_NITEM, _D), jnp.float32),
    mesh=_mesh,
    compiler_params=pltpu.CompilerParams(needs_layout_passes=False),
    scratch_types=[
        pltpu.VMEM((_MMW, 32), jnp.int32),
        pltpu.VMEM((_MMW, 32), jnp.float32),
        pltpu.VMEM((20, _D), jnp.float32),
        pltpu.VMEM((20, _D), jnp.float32),
        pltpu.VMEM((20, _D), jnp.float32),
        pltpu.VMEM((20, _D), jnp.float32),
        pltpu.VMEM((_MMW, _D), jnp.float32),
        pltpu.SemaphoreType.DMA,
        pltpu.SemaphoreType.DMA,
        pltpu.SemaphoreType.DMA,
        pltpu.SemaphoreType.DMA,
    ],
)
def _mm_prop(h_hbm, cols_hbm, vals_hbm, out_hbm, *rest):
    _mm_body(h_hbm, cols_hbm, vals_hbm, out_hbm, *rest)


_MCH = 32
_MNCH = _EPAD // _MCH


def _mask_deg_body(ego_hbm, ss_hbm, eu_hbm, ei_hbm, mask_hbm, deg_hbm,
                   eu_v, eip_v, mask_v, ss_v, degp_v, bufU0, bufI0,
                   bufU1, bufI1, s_v, red_v, degout_v, shared,
                   semU0, semI0, semU1, semI1):
    c = lax.axis_index("c")
    s = lax.axis_index("s")
    wid = s * _NC + c
    base = pl.multiple_of(wid * _EW, 8)
    zi = jnp.zeros((16,), jnp.int32)
    zf = jnp.zeros((16,), jnp.float32)
    for t in range(8):
        off = _EPAD - 128 + t * 16
        eu_v[pl.ds(off, 16)] = zi
        eip_v[pl.ds(off, 16)] = zi
    pltpu.sync_copy(eu_hbm.at[pl.ds(base, _EW)], eu_v.at[pl.ds(0, _EW)])
    pltpu.sync_copy(ei_hbm.at[pl.ds(base, _EW)], eip_v.at[pl.ds(0, _EW)])
    pltpu.sync_copy(ss_hbm, ss_v)

    def zdeg(k, _):
        degp_v[pl.ds(k * 16, 16)] = zf
        return 0

    lax.fori_loop(0, _PK // 16, zdeg, 0)

    def addoff(k, _):
        sl = pl.ds(k * 16, 16)
        eip_v[sl] = eip_v[sl] + _NUSER
        return 0

    lax.fori_loop(0, _EPAD // 16, addoff, 0)
    lanes = lax.iota(jnp.int32, 16)
    l16 = lanes * 16

    def gather(j, bufU, bufI, semU, semI):
        j = jnp.minimum(j, _MNCH - 1)
        e0 = pl.multiple_of(j * _MCH, _MCH)
        pltpu.async_copy(ego_hbm.at[eu_v.at[pl.ds(e0, _MCH)]], bufU, semU)
        pltpu.async_copy(ego_hbm.at[eip_v.at[pl.ds(e0, _MCH)]], bufI, semI)

    def drain(bufU, bufI, semU, semI):
        pltpu.make_async_copy(ego_hbm.at[eu_v.at[pl.ds(0, _MCH)]],
                              bufU, semU).wait()
        pltpu.make_async_copy(ego_hbm.at[eip_v.at[pl.ds(0, _MCH)]],
                              bufI, semI).wait()

    def process(j, bufU, bufI):
        e0 = pl.multiple_of(j * _MCH, _MCH)
        for k in range(_MCH // 16):
            for l in range(16):
                el = k * 16 + l
                p = [bufU[el, pl.ds(g * 16, 16)] * bufI[el, pl.ds(g * 16, 16)]
                     for g in range(8)]
                while len(p) > 1:
                    p = [p[i] + p[i + 1] for i in range(0, len(p), 2)]
                s_v[pl.ds(l * 16, 16)] = p[0]
            q = [plsc.load_gather(s_v, [l16 + jj]) for jj in range(16)]
            while len(q) > 1:
                q = [q[i] + q[i + 1] for i in range(0, len(q), 2)]
            dvec = q[0]
            o = e0 + k * 16
            eu = eu_v[pl.ds(o, 16)]
            eip = eip_v[pl.ds(o, 16)]
            ssu = plsc.load_gather(ss_v, [eu])
            ssi = plsc.load_gather(ss_v, [eip + (_IOFF - _NUSER)])
            m = (dvec >= 0.0) & (dvec * dvec >= 0.0025 * ssu * ssi)
            m = m & (o + lanes < _EW)
            mf = jnp.where(m, 1.0, 0.0)
            mask_v[pl.ds(o, 16)] = mf
            plsc.addupdate_scatter(degp_v, [eu], mf)
            plsc.addupdate_scatter(degp_v, [eip + (_IOFF - _NUSER)], mf)

    gather(0, bufU0, bufI0, semU0, semI0)

    def pair(t, _):
        j0 = 2 * t
        gather(j0 + 1, bufU1, bufI1, semU1, semI1)
        drain(bufU0, bufI0, semU0, semI0)
        process(j0, bufU0, bufI0)
        gather(j0 + 2, bufU0, bufI0, semU0, semI0)
        drain(bufU1, bufI1, semU1, semI1)
        process(j0 + 1, bufU1, bufI1)
        return 0

    lax.fori_loop(0, _MNCH // 2, pair, 0)
    drain(bufU0, bufI0, semU0, semI0)
    pltpu.sync_copy(mask_v.at[pl.ds(0, _EW)], mask_hbm.at[pl.ds(base, _EW)])
    pltpu.sync_copy(degp_v, shared.at[s])
    plsc.subcore_barrier()
    col0 = pl.multiple_of(s * _COLS_PER_SUB, 128)
    pltpu.sync_copy(shared.at[pl.ds(0, _NS), pl.ds(col0, _COLS_PER_SUB)], red_v)

    def red(k, _):
        sl = pl.ds(k * 16, 16)
        a = red_v[0, sl]
        for t2 in range(1, _NS):
            a = a + red_v[t2, sl]
        degout_v[sl] = a
        return 0

    lax.fori_loop(0, _COLS_PER_SUB // 16, red, 0)
    pltpu.sync_copy(
        degout_v,
        deg_hbm.at[pl.ds(pl.multiple_of(c * _PK + col0, 128), _COLS_PER_SUB)])


@functools.partial(
    pl.kernel,
    out_type=[
        jax.ShapeDtypeStruct((_E + _CH,), jnp.float32),
        jax.ShapeDtypeStruct((_NC * _PK,), jnp.float32),
    ],
    mesh=_mesh,
    compiler_params=pltpu.CompilerParams(needs_layout_passes=False),
    scratch_types=[
        pltpu.VMEM((_EPAD,), jnp.int32),
        pltpu.VMEM((_EPAD,), jnp.int32),
        pltpu.VMEM((_EPAD,), jnp.float32),
        pltpu.VMEM((_PK,), jnp.float32),
        pltpu.VMEM((_PK,), jnp.float32),
        pltpu.VMEM((_MCH, _D), jnp.float32),
        pltpu.VMEM((_MCH, _D), jnp.float32),
        pltpu.VMEM((_MCH, _D), jnp.float32),
        pltpu.VMEM((_MCH, _D), jnp.float32),
        pltpu.VMEM((256,), jnp.float32),
        pltpu.VMEM((_NS, _COLS_PER_SUB), jnp.float32),
        pltpu.VMEM((_COLS_PER_SUB,), jnp.float32),
        pltpu.VMEM_SHARED((_NS, _PK), jnp.float32),
        pltpu.SemaphoreType.DMA,
        pltpu.SemaphoreType.DMA,
        pltpu.SemaphoreType.DMA,
        pltpu.SemaphoreType.DMA,
    ],
)
def _mask_deg(ego_hbm, ss_hbm, eu_hbm, ei_hbm, mask_hbm, deg_hbm, *rest):
    _mask_deg_body(ego_hbm, ss_hbm, eu_hbm, ei_hbm, mask_hbm, deg_hbm, *rest)


def _gcn_scatter_body(y_hbm, eu_hbm, ei_hbm, mask_hbm, zeros_hbm, out_hbm,
                      eubuf, eibuf, mbuf, srcu_v, srci_v, d1_v, d2_v,
                      bufA, bufB, acc_sh, semA, semB, semSA, semSB):
    c = lax.axis_index("c")
    s = lax.axis_index("s")
    wid = s * _NC + c
    base = pl.multiple_of(wid * _EW, 8)
    rows0 = pl.multiple_of(s * _ROWS_PER_SUB, 8)
    pltpu.sync_copy(zeros_hbm.at[pl.ds(rows0, _ROWS_PER_SUB)],
                    acc_sh.at[pl.ds(rows0, _ROWS_PER_SUB)])
    lanes = lax.iota(jnp.int32, 16)

    def build(j, slot):
        j = jnp.minimum(j, _NCH - 1)
        e0 = pl.multiple_of(j * _CH, _CH)
        pltpu.sync_copy(mask_hbm.at[pl.ds(base + e0, _CH)], mbuf)
        pltpu.sync_copy(eu_hbm.at[pl.ds(base + e0, _CH)], eubuf)
        pltpu.sync_copy(ei_hbm.at[pl.ds(base + e0, _CH)], eibuf)
        for k in range(_CH // 16):
            sl = pl.ds(k * 16, 16)
            eu = eubuf[sl]
            ei = eibuf[sl] + _NUSER
            m = (mbuf[sl] > 0.5) & (e0 + k * 16 + lanes < _EW)
            pad = _N + ((wid + k) * 16) % 112 + lanes
            srcu_v[slot, sl] = jnp.where(m, eu, pad)
            srci_v[slot, sl] = jnp.where(m, ei, pad)
            d1_v[slot, sl] = ei
            d2_v[slot, sl] = eu

    def gathers(slot):
        pltpu.async_copy(y_hbm.at[srcu_v.at[slot]], bufA, semA)
        pltpu.async_copy(y_hbm.at[srci_v.at[slot]], bufB, semB)

    def gwait():
        pltpu.make_async_copy(y_hbm.at[srcu_v.at[0]], bufA, semA).wait()
        pltpu.make_async_copy(y_hbm.at[srci_v.at[0]], bufB, semB).wait()

    def scatters(slot):
        pltpu.async_copy(bufA, acc_sh.at[d1_v.at[slot]], semSA, add=True)
        pltpu.async_copy(bufB, acc_sh.at[d2_v.at[slot]], semSB, add=True)

    def swait():
        pltpu.make_async_copy(bufA, acc_sh.at[d1_v.at[0]], semSA).wait()
        pltpu.make_async_copy(bufB, acc_sh.at[d2_v.at[0]], semSB).wait()

    plsc.subcore_barrier()
    build(0, 0)
    gathers(0)

    def step2(t, _):
        j0 = 2 * t
        build(j0 + 1, 1)
        gwait()
        scatters(0)
        swait()
        gathers(1)
        build(j0 + 2, 0)
        gwait()
        scatters(1)
        swait()
        gathers(0)
        return 0

    lax.fori_loop(0, _NCH // 2, step2, 0)
    gwait()
    plsc.subcore_barrier()
    pltpu.sync_copy(acc_sh.at[pl.ds(rows0, _ROWS_PER_SUB)],
                    out_hbm.at[c, pl.ds(rows0, _ROWS_PER_SUB)])


@functools.partial(
    pl.kernel,
    out_type=jax.ShapeDtypeStruct((_NC, _YPAD, _D), jnp.float32),
    mesh=_mesh,
    compiler_params=pltpu.CompilerParams(needs_layout_passes=False),
    scratch_types=[
        pltpu.VMEM((_CH,), jnp.int32),
        pltpu.VMEM((_CH,), jnp.int32),
        pltpu.VMEM((_CH,), jnp.float32),
        pltpu.VMEM((2, _CH), jnp.int32),
        pltpu.VMEM((2, _CH), jnp.int32),
        pltpu.VMEM((2, _CH), jnp.int32),
        pltpu.VMEM((2, _CH), jnp.int32),
        pltpu.VMEM((_CH, _D), jnp.float32),
        pltpu.VMEM((_CH, _D), jnp.float32),
        pltpu.VMEM_SHARED((_YPAD, _D), jnp.float32),
        pltpu.SemaphoreType.DMA,
        pltpu.SemaphoreType.DMA,
        pltpu.SemaphoreType.DMA,
        pltpu.SemaphoreType.DMA,
    ],
)
def _gcn_scatter(y_hbm, eu_hbm, ei_hbm, mask_hbm, zeros_hbm, out_hbm, *rest):
    _gcn_scatter_body(y_hbm, eu_hbm, ei_hbm, mask_hbm, zeros_hbm, out_hbm,
                      *rest)


def _gcn_layer(ego, W, b, e_user, e_item, eu_pad, ei_pad, zeros_pad):
    xw, ss = _mm_ss(ego, W)
    ss1 = ss[:, 0]
    ss_packed = jnp.concatenate([
        ss1[:_NUSER], jnp.ones((_IOFF - _NUSER,), jnp.float32),
        ss1[_NUSER:], jnp.ones((_PK - _IOFF - _NITEM,), jnp.float32),
    ])
    mask, degp_flat = _mask_deg(ego, ss_packed, e_user, e_item)
    degp = degp_flat.reshape(_NC, _PK)
    degsum = degp[0] + degp[1]
    deg2d = jnp.concatenate(
        [degsum[:_NUSER], degsum[_IOFF:_IOFF + _NITEM]]).reshape(_N, 1)
    y, selfo, dis = _yprep(deg2d, xw, b)
    y_full = jnp.concatenate([y, jnp.zeros((_YPAD - _N, _D), jnp.float32)], 0)
    acc = _gcn_scatter(y_full, eu_pad, ei_pad, mask, zeros_pad)
    return _combine(dis, acc[0, :_N], acc[1, :_N], selfo)


def kernel(user_emb, item_emb, img_W, img_b, txt_W, txt_b, Wg0, bg0, Wg1, bg1,
           Wm0, bm0, Wm1, bm1, v_feat, t_feat, mm_vals, e_user, e_item,
           mm_rows, mm_cols):
    colsA = mm_cols[:_NITEM * 10].reshape(_NITEM, 10)
    colsB = mm_cols[_NITEM * 10:].reshape(_NITEM, 10)
    valsA = mm_vals[:_NITEM * 10].reshape(_NITEM, 10)
    valsB = mm_vals[_NITEM * 10:].reshape(_NITEM, 10)
    spread = (jnp.arange(_NITEM, dtype=jnp.int32)[:, None]
              + jnp.arange(12, dtype=jnp.int32)[None, :]) % _NITEM
    cols32 = jnp.concatenate([colsA, colsB, spread], axis=1)
    vals32 = jnp.concatenate(
        [valsA, valsB, jnp.zeros((_NITEM, 12), jnp.float32)], axis=1)
    h = item_emb
    h = _mm_prop(h, cols32, vals32)
    h = _mm_prop(h, cols32, vals32)

    zeros_pad = jnp.zeros((_YPAD, _D), jnp.float32)
    eu_pad = jnp.concatenate([e_user, jnp.zeros((256,), jnp.int32)])
    ei_pad = jnp.concatenate([e_item, jnp.zeros((256,), jnp.int32)])
    ego = jnp.concatenate([user_emb, item_emb], 0)
    alls = ego
    for (W, b) in ((Wg0, bg0), (Wg1, bg1)):
        ego = _gcn_layer(ego, W, b, e_user, e_item, eu_pad, ei_pad,
                         zeros_pad)
        alls = alls + ego
    return _final(alls, h)
